# baseline (speedup 1.0000x reference)
import functools
import math

import jax
import jax.numpy as jnp
from jax.experimental import pallas as pl
from jax.experimental.pallas import tpu as pltpu

D_MODEL = 768
N_HEAD = 12
D_HEAD = 64
N_EXP = 64
S_LEN = 2048
S_BLK = 512
EMA_DECAY = 0.99


def _head_kernel(xf_ref, xh_ref, wq_ref, wk_ref, wr_ref, wv_ref, wo_ref,
                 ema_ref, y_ref, ema_out_ref):
    f32 = jnp.float32
    xf = xf_ref[...]
    xh = xh_ref[0]
    wq = wq_ref[...]
    wk = wk_ref[...]
    wr = wr_ref[0]
    wv = wv_ref[0]
    wo = wo_ref[0]

    dims = (((1,), (1,)), ((), ()))
    q = jax.lax.dot_general(xf, wq, dims, preferred_element_type=f32)
    k = jax.lax.dot_general(xf, wk, dims, preferred_element_type=f32)
    gl = jax.lax.dot_general(xf, wr, dims, preferred_element_type=f32)

    gmax = jnp.max(gl, axis=1, keepdims=True)
    lane_e = jax.lax.broadcasted_iota(jnp.int32, (S_LEN, N_EXP), 1)
    idx = jnp.min(jnp.where(gl == gmax, lane_e, N_EXP), axis=1,
                  keepdims=True)

    cnt = jnp.sum((idx == lane_e).astype(f32), axis=0, keepdims=True)
    ema = ema_ref[0] * EMA_DECAY + cnt * ((1.0 - EMA_DECAY) / S_LEN)
    ema_out_ref[0] = ema

    scale = 1.0 / math.sqrt(D_HEAD)
    nb = S_LEN // S_BLK
    lane_blk = jax.lax.broadcasted_iota(jnp.int32, (S_BLK, N_EXP * D_HEAD), 1) >> 6

    attn = jnp.zeros((S_LEN, D_HEAD), f32)
    for b in range(nb):
        lo, hi = b * S_BLK, (b + 1) * S_BLK
        qb = q[lo:hi, :]
        st = jax.lax.dot_general(k, qb, dims, preferred_element_type=f32)
        st = st * scale
        s_glob = b * S_BLK + jax.lax.broadcasted_iota(jnp.int32, (S_LEN, S_BLK), 1)
        t_row = jax.lax.broadcasted_iota(jnp.int32, (S_LEN, S_BLK), 0)
        st = st + jnp.where(t_row <= s_glob, 0.0, -1e9)
        cm = jnp.max(st, axis=0, keepdims=True)
        p = jnp.exp(st - cm)
        p = p / jnp.sum(p, axis=0, keepdims=True)

        xb = xh[lo:hi, :].astype(jnp.bfloat16)
        xe = jnp.tile(xb, (1, N_EXP))
        xs = jnp.where(lane_blk == idx[lo:hi, :], xe, jnp.bfloat16(0))
        vb = jax.lax.dot_general(xs, wv, (((1,), (0,)), ((), ())),
                                 preferred_element_type=f32)
        attn = attn + jax.lax.dot_general(
            p.astype(jnp.bfloat16), vb.astype(jnp.bfloat16),
            (((1,), (0,)), ((), ())), preferred_element_type=f32)

    for b in range(nb):
        lo, hi = b * S_BLK, (b + 1) * S_BLK
        ab = attn[lo:hi, :].astype(jnp.bfloat16)
        ae = jnp.tile(ab, (1, N_EXP))
        as_ = jnp.where(lane_blk == idx[lo:hi, :], ae, jnp.bfloat16(0))
        yb = jax.lax.dot_general(as_, wo, (((1,), (0,)), ((), ())),
                                 preferred_element_type=f32)
        y_ref[0, lo:hi, :] = yb


@functools.partial(jax.jit, static_argnames=())
def kernel(x, mask, W_q, W_k, W_v, W_o, router_W, ema_counts):
    del mask
    B, S, D = x.shape
    h, E, d_h = N_HEAD, N_EXP, D_HEAD

    wv_flat = W_v.reshape(h, E * d_h, d_h).astype(jnp.bfloat16)
    wo_flat = W_o.reshape(h, E * d_h, d_h).astype(jnp.bfloat16)
    wr3 = router_W.reshape(h, E, D)
    ema3 = ema_counts.reshape(h, 1, E)
    x2 = x.reshape(S, D)
    x_heads = x2.reshape(S, h, d_h).transpose(1, 0, 2)

    grid = (h,)
    yh, ema = pl.pallas_call(
        _head_kernel,
        grid=grid,
        in_specs=[
            pl.BlockSpec((S, D), lambda i: (0, 0)),
            pl.BlockSpec((1, S, d_h), lambda i: (i, 0, 0)),
            pl.BlockSpec((d_h, D), lambda i: (i, 0)),
            pl.BlockSpec((d_h, D), lambda i: (i, 0)),
            pl.BlockSpec((1, E, D), lambda i: (i, 0, 0)),
            pl.BlockSpec((1, E * d_h, d_h), lambda i: (i, 0, 0)),
            pl.BlockSpec((1, E * d_h, d_h), lambda i: (i, 0, 0)),
            pl.BlockSpec((1, 1, E), lambda i: (i, 0, 0)),
        ],
        out_specs=[
            pl.BlockSpec((1, S, d_h), lambda i: (i, 0, 0)),
            pl.BlockSpec((1, 1, E), lambda i: (i, 0, 0)),
        ],
        out_shape=[
            jax.ShapeDtypeStruct((h, S, d_h), jnp.float32),
            jax.ShapeDtypeStruct((h, 1, E), jnp.float32),
        ],
    )(x2, x_heads, W_q, W_k, wr3, wv_flat, wo_flat, ema3)

    y = yh.transpose(1, 0, 2).reshape(1, S, D)
    ema2 = ema.reshape(h, E)
    lb_loss = (ema2 * ema2).sum() * (E * h) / jnp.square(ema2.sum() + 1e-9)
    return (y, lb_loss)


# speedup vs baseline: 1.3428x; 1.3428x over previous
import functools
import math

import jax
import jax.numpy as jnp
from jax.experimental import pallas as pl
from jax.experimental.pallas import tpu as pltpu

D_MODEL = 768
N_HEAD = 12
D_HEAD = 64
N_EXP = 64
S_LEN = 2048
S_BLK = 512
EMA_DECAY = 0.99


def _head_kernel(xf_ref, wq_ref, wk_ref, wr_ref, wv_ref, wo_ref,
                 ema_ref, y_ref, ema_out_ref):
    f32 = jnp.float32
    i = pl.program_id(0)
    xf = xf_ref[...]
    wq = wq_ref[...]
    wk = wk_ref[...]
    wr = wr_ref[0]
    wv = wv_ref[0]
    wo = wo_ref[0]

    dims = (((1,), (1,)), ((), ()))
    q = jax.lax.dot_general(xf, wq, dims, preferred_element_type=f32)
    k = jax.lax.dot_general(xf, wk, dims, preferred_element_type=f32)
    gl = jax.lax.dot_general(xf, wr, dims, preferred_element_type=f32)

    xpair = xf_ref[:, pl.ds((i >> 1) * 2 * D_HEAD, 2 * D_HEAD)]
    xh = jnp.where((i & 1) == 0, xpair[:, :D_HEAD], xpair[:, D_HEAD:])

    gmax = jnp.max(gl, axis=1, keepdims=True)
    lane_e = jax.lax.broadcasted_iota(jnp.int32, (S_LEN, N_EXP), 1)
    idx = jnp.min(jnp.where(gl == gmax, lane_e, N_EXP), axis=1,
                  keepdims=True)

    cnt = jnp.sum((idx == lane_e).astype(f32), axis=0, keepdims=True)
    ema = ema_ref[0] * EMA_DECAY + cnt * ((1.0 - EMA_DECAY) / S_LEN)
    ema_out_ref[0] = ema

    scale = 1.0 / math.sqrt(D_HEAD)
    nb = S_LEN // S_BLK
    lane_blk = jax.lax.broadcasted_iota(jnp.int32, (S_BLK, N_EXP * D_HEAD), 1) >> 6

    attn = jnp.zeros((S_LEN, D_HEAD), f32)
    for b in range(nb):
        lo, hi = b * S_BLK, (b + 1) * S_BLK
        qb = q[lo:hi, :]
        st = jax.lax.dot_general(k, qb, dims, preferred_element_type=f32)
        st = st * scale
        s_glob = b * S_BLK + jax.lax.broadcasted_iota(jnp.int32, (S_LEN, S_BLK), 1)
        t_row = jax.lax.broadcasted_iota(jnp.int32, (S_LEN, S_BLK), 0)
        st = st + jnp.where(t_row <= s_glob, 0.0, -1e9)
        cm = jnp.max(st, axis=0, keepdims=True)
        p = jnp.exp(st - cm)
        p = p / jnp.sum(p, axis=0, keepdims=True)

        xb = xh[lo:hi, :]
        xe = jnp.tile(xb, (1, N_EXP))
        xs = jnp.where(lane_blk == idx[lo:hi, :], xe, 0.0)
        vb = jax.lax.dot_general(xs, wv, (((1,), (0,)), ((), ())),
                                 preferred_element_type=f32)
        attn = attn + jax.lax.dot_general(p, vb, (((1,), (0,)), ((), ())),
                                          preferred_element_type=f32)

    lane128 = jax.lax.broadcasted_iota(jnp.int32, (S_BLK, 2 * D_HEAD), 1)
    mine = (lane128 >> 6) == (i & 1)
    for b in range(nb):
        lo, hi = b * S_BLK, (b + 1) * S_BLK
        ab = attn[lo:hi, :]
        ae = jnp.tile(ab, (1, N_EXP))
        as_ = jnp.where(lane_blk == idx[lo:hi, :], ae, 0.0)
        yb = jax.lax.dot_general(as_, wo, (((1,), (0,)), ((), ())),
                                 preferred_element_type=f32)
        y_ref[lo:hi, :] = jnp.where(mine, jnp.tile(yb, (1, 2)),
                                    y_ref[lo:hi, :])


@functools.partial(jax.jit, static_argnames=())
def kernel(x, mask, W_q, W_k, W_v, W_o, router_W, ema_counts):
    del mask
    B, S, D = x.shape
    h, E, d_h = N_HEAD, N_EXP, D_HEAD

    wv_flat = W_v.reshape(h, E * d_h, d_h)
    wo_flat = W_o.reshape(h, E * d_h, d_h)
    wr3 = router_W.reshape(h, E, D)
    ema3 = ema_counts.reshape(h, 1, E)
    x2 = x.reshape(S, D)

    grid = (h,)
    y2, ema = pl.pallas_call(
        _head_kernel,
        grid=grid,
        in_specs=[
            pl.BlockSpec((S, D), lambda i: (0, 0)),
            pl.BlockSpec((d_h, D), lambda i: (i, 0)),
            pl.BlockSpec((d_h, D), lambda i: (i, 0)),
            pl.BlockSpec((1, E, D), lambda i: (i, 0, 0)),
            pl.BlockSpec((1, E * d_h, d_h), lambda i: (i, 0, 0)),
            pl.BlockSpec((1, E * d_h, d_h), lambda i: (i, 0, 0)),
            pl.BlockSpec((1, 1, E), lambda i: (i, 0, 0)),
        ],
        out_specs=[
            pl.BlockSpec((S, 2 * d_h), lambda i: (0, i // 2)),
            pl.BlockSpec((1, 1, E), lambda i: (i, 0, 0)),
        ],
        out_shape=[
            jax.ShapeDtypeStruct((S, D), jnp.float32),
            jax.ShapeDtypeStruct((h, 1, E), jnp.float32),
        ],
    )(x2, W_q, W_k, wr3, wv_flat, wo_flat, ema3)

    y = y2.reshape(1, S, D)
    ema2 = ema.reshape(h, E)
    lb_loss = (ema2 * ema2).sum() * (E * h) / jnp.square(ema2.sum() + 1e-9)
    return (y, lb_loss)
